# baseline (speedup 1.0000x reference)
import functools

import jax
import jax.numpy as jnp
from jax import lax
from jax.experimental import pallas as pl
from jax.experimental.pallas import tpu as pltpu
from jax.experimental.pallas import tpu_sc as plsc

_HI_BINS = (8, 16, 32, 64)
_DIM = 128
_NUM_EMB = 9
_B = 16384
_NC = 2
_NS = 16
_NW = _NC * _NS
_B_PER_W = _B // _NW
_CHUNK = 64
_NCHUNK = _B_PER_W // _CHUNK

_mesh = plsc.VectorSubcoreMesh(core_axis_name="c", subcore_axis_name="s")


@functools.partial(
    pl.kernel,
    out_type=jax.ShapeDtypeStruct((_B, _DIM), jnp.float32),
    mesh=_mesh,
    scratch_types=[
        pltpu.VMEM((_NUM_EMB, _DIM), jnp.float32),
        pltpu.VMEM_SHARED((_NUM_EMB, _DIM), jnp.float32),
        pltpu.VMEM((_B_PER_W,), jnp.int32),
        pltpu.VMEM((_NCHUNK, _CHUNK), jnp.int32),
        pltpu.VMEM((_B_PER_W, _DIM), jnp.float32),
        pltpu.SemaphoreType.DMA,
        pltpu.SemaphoreType.DMA,
        pltpu.SemaphoreType.DMA,
        pltpu.SemaphoreType.DMA,
        pltpu.SemaphoreType.DMA,
        pltpu.SemaphoreType.DMA,
        pltpu.SemaphoreType.DMA,
        pltpu.SemaphoreType.DMA,
        pltpu.SemaphoreType.DMA,
        pltpu.SemaphoreType.DMA,
        pltpu.SemaphoreType.DMA,
    ],
)
def _distance_sc(lengths_hbm, table_hbm, out_hbm,
                 tab_v, tab_sh, len_v, idx_v, rows_v,
                 tsem, g0, g1, g2, g3, g4, g5, g6, g7, osem, lsem):
    gsems = (g0, g1, g2, g3, g4, g5, g6, g7)
    sid = lax.axis_index("s")
    wid = sid * _NC + lax.axis_index("c")
    base = wid * _B_PER_W

    @pl.when(sid == 0)
    def _():
        pltpu.async_copy(table_hbm, tab_v, tsem)

    tail = pltpu.async_copy(
        lengths_hbm.at[pl.ds(base + _CHUNK, _B_PER_W - _CHUNK)],
        len_v.at[pl.ds(_CHUNK, _B_PER_W - _CHUNK)], lsem)
    pltpu.sync_copy(lengths_hbm.at[pl.ds(base, _CHUNK)],
                    len_v.at[pl.ds(0, _CHUNK)])

    ones = jnp.full((16,), 1, jnp.int32)
    zeros = jnp.full((16,), 0, jnp.int32)
    four = jnp.full((16,), 4, jnp.int32)

    def compute_chunk(j):
        for k in range(_CHUNK // 16):
            v = len_v[pl.ds(j * _CHUNK + k * 16, 16)]
            acc = jnp.minimum(jnp.maximum(v, zeros), four)
            for b in _HI_BINS:
                acc = acc + jnp.where(v >= jnp.full((16,), b, jnp.int32),
                                      ones, zeros)
            idx_v[j, pl.ds(k * 16, 16)] = acc

    compute_chunk(0)

    @pl.when(sid == 0)
    def _():
        pltpu.make_async_copy(table_hbm, tab_v, tsem).wait()
        pltpu.sync_copy(tab_v, tab_sh)

    tail.wait()
    plsc.subcore_barrier()

    gathers = []
    for j in range(_NCHUNK):
        gathers.append(
            pltpu.async_copy(tab_sh.at[idx_v.at[j]],
                             rows_v.at[pl.ds(j * _CHUNK, _CHUNK)], gsems[j]))
        if j + 1 < _NCHUNK:
            compute_chunk(j + 1)

    outs = []
    for j in range(_NCHUNK):
        gathers[j].wait()
        outs.append(
            pltpu.async_copy(
                rows_v.at[pl.ds(j * _CHUNK, _CHUNK)],
                out_hbm.at[pl.ds(base + j * _CHUNK, _CHUNK)],
                osem))
    for c in outs:
        c.wait()


def kernel(lengths, table):
    return _distance_sc(lengths, table)
